# baseline (speedup 1.0000x reference)
import functools

import jax
import jax.numpy as jnp
from jax import lax
from jax.experimental import pallas as pl
from jax.experimental.pallas import tpu as pltpu
from jax.experimental.pallas import tpu_sc as plsc

B = 1024
L = 200
EMBED_DIM = 64
NUM_PAIRS = B * L
NUM_ROWS = 2 * NUM_PAIRS

NUM_CORES = 2
NUM_SUBCORES = 16
NUM_WORKERS = NUM_CORES * NUM_SUBCORES
LANES = 16

PAIRS_PER_WORKER = NUM_PAIRS // NUM_WORKERS
CHUNK_PAIRS = 256
CHUNK_ROWS = 2 * CHUNK_PAIRS
NUM_CHUNKS = PAIRS_PER_WORKER // CHUNK_PAIRS
GATHER_GROUPS = CHUNK_ROWS // 128


def _emb_lookup(trip_flat, emb_table):
    mesh = plsc.VectorSubcoreMesh(core_axis_name="c", subcore_axis_name="s")

    @functools.partial(
        pl.kernel,
        mesh=mesh,
        out_type=jax.ShapeDtypeStruct((NUM_ROWS, EMBED_DIM), jnp.float32),
        compiler_params=pltpu.CompilerParams(
            needs_layout_passes=False, use_tc_tiling_on_sc=False
        ),
        scratch_types=[
            pltpu.VMEM((3 * CHUNK_PAIRS,), jnp.int32),
            pltpu.VMEM((GATHER_GROUPS, 128), jnp.int32),
            pltpu.VMEM((CHUNK_ROWS, EMBED_DIM), jnp.float32),
            pltpu.SemaphoreType.DMA,
        ],
    )
    def k(trip_hbm, table_hbm, out_hbm, trip_v, idx_v, rows_v, sem):
        wid = lax.axis_index("s") * NUM_CORES + lax.axis_index("c")
        pair_base = wid * PAIRS_PER_WORKER

        lane = lax.iota(jnp.int32, LANES)

        def chunk_body(g, _):
            m0 = pair_base + g * CHUNK_PAIRS
            pltpu.sync_copy(
                trip_hbm.at[pl.ds(3 * m0, 3 * CHUNK_PAIRS)], trip_v
            )

            copies = []
            for j in range(GATHER_GROUPS):
                def extract(v8, _, j=j):
                    k0 = (j * 8 + v8) * LANES
                    kk = k0 + lane
                    src = (kk >> 1) * 3 + (kk & 1) * 2
                    idx16 = plsc.load_gather(trip_v, [src])
                    idx_v[j, pl.ds(v8 * LANES, LANES)] = idx16
                    return _

                lax.fori_loop(0, 8, extract, None)
                copies.append(
                    pltpu.async_copy(
                        table_hbm.at[idx_v.at[j]],
                        rows_v.at[pl.ds(j * 128, 128)],
                        sem,
                    )
                )
            for c in copies:
                c.wait()

            pltpu.sync_copy(
                rows_v, out_hbm.at[pl.ds(2 * m0, CHUNK_ROWS)]
            )
            return _

        lax.fori_loop(0, NUM_CHUNKS, chunk_body, None)

    return k(trip_flat, emb_table)


def kernel(triples, emb_table):
    trip_flat = triples.reshape(-1)
    out = _emb_lookup(trip_flat, emb_table)
    return out.reshape(B, L, 2, EMBED_DIM)
